# Initial kernel scaffold; baseline (speedup 1.0000x reference)
import jax
import jax.numpy as jnp
from jax.experimental import pallas as pl


def kernel(Di, DiA, mask, inputs, W1, b1, rn0_W0, rn0_b0, rn0_W1, rn0_b1, rn1_W0, rn1_b0, rn1_W1, rn1_b1, rn2_W0, rn2_b0, rn2_W1, rn2_b1, rn3_W0, rn3_b0, rn3_W1, rn3_b1, W2, b2):
    raise NotImplementedError("write your pallas kernel here")


# speedup vs baseline: 1.2074x; 1.2074x over previous
import functools

import jax
import jax.numpy as jnp
from jax.experimental import pallas as pl


def _elu(x):
    return jnp.where(x > 0, x, jnp.exp(x) - 1.0)


def _bn(x):
    mu = jnp.mean(x, axis=0, keepdims=True)
    var = jnp.mean((x - mu) ** 2, axis=0, keepdims=True)
    return (x - mu) * jax.lax.rsqrt(var + 1e-5)


def _dot(a, b):
    return jax.lax.dot_general(a, b, (((1,), (0,)), ((), ())),
                               preferred_element_type=jnp.float32)


def _avg_block(v, w0, b0, w1, b1):
    h = _dot(_bn(_elu(v)), w0) + b0
    return v + _dot(_bn(_elu(h)), w1) + b1


def _pre_body(inp_ref, w1_ref, b1_ref, w0t_ref, b0_ref, a0_ref, a1_ref,
              a2_ref, a3_ref, xin_ref, v2_ref):
    inp = inp_ref[...]
    w1 = w1_ref[...]
    v0 = (inp[:, 0:1] * w1[0:1, :] + inp[:, 1:2] * w1[1:2, :]
          + inp[:, 2:3] * w1[2:3, :] + b1_ref[...])
    x_in = _elu(v0)
    v1 = v0 + _dot(_bn(x_in), w0t_ref[...]) + b0_ref[...]
    v2 = _avg_block(v1, a0_ref[...], a1_ref[...], a2_ref[...], a3_ref[...])
    xin_ref[...] = x_in
    v2_ref[...] = v2


def _mid_body(dv_ref, w_ref, b_ref, fq_ref):
    fq_ref[...] = _elu(_dot(_bn(dv_ref[...]), w_ref[...]) + b_ref[...])


def _post_body(v2_ref, daf_ref, w2a_ref, w2b_ref, b2_ref, a0_ref, a1_ref,
               a2_ref, a3_ref, wo_ref, bo_ref, tail_ref, out_ref):
    v2 = v2_ref[...]
    x = _dot(_bn(_elu(v2)), w2a_ref[...]) + _dot(_bn(daf_ref[...]), w2b_ref[...])
    v3 = v2 + x + b2_ref[...]
    v4 = _avg_block(v3, a0_ref[...], a1_ref[...], a2_ref[...], a3_ref[...])
    out_ref[...] = _dot(_bn(_elu(v4)), wo_ref[...]) + bo_ref[...] + tail_ref[...]


def _mm_body(m_ref, x_ref, o_ref):
    o_ref[...] = _dot(m_ref[...], x_ref[...])


def _stream_matmul(m, x, block_rows):
    r, k = m.shape
    return pl.pallas_call(
        _mm_body,
        grid=(r // block_rows,),
        in_specs=[
            pl.BlockSpec((block_rows, k), lambda i: (i, 0)),
            pl.BlockSpec((k, x.shape[1]), lambda i: (0, 0)),
        ],
        out_specs=pl.BlockSpec((block_rows, x.shape[1]), lambda i: (i, 0)),
        out_shape=jax.ShapeDtypeStruct((r, x.shape[1]), jnp.float32),
    )(m, x)


def _full_call(body, n_out_rows, n_out_cols, args, n_outs=1):
    shapes = [jax.ShapeDtypeStruct((nr, nc), jnp.float32)
              for nr, nc in zip(n_out_rows, n_out_cols)]
    return pl.pallas_call(
        body,
        out_shape=shapes if n_outs > 1 else shapes[0],
    )(*args)


def kernel(Di, DiA, mask, inputs, W1, b1, rn0_W0, rn0_b0, rn0_W1, rn0_b1,
           rn1_W0, rn1_b0, rn1_W1, rn1_b1, rn2_W0, rn2_b0, rn2_W1, rn2_b1,
           rn3_W0, rn3_b0, rn3_W1, rn3_b1, W2, b2):
    del mask
    n = inputs.shape[1]
    fc = Di.shape[1] // 4
    inp = inputs[0]
    r1 = lambda v: v.reshape(1, -1)

    x_in, v2 = _full_call(
        _pre_body, (n, n), (128, 128),
        (inp, W1, r1(b1), rn0_W0[:128], r1(rn0_b0),
         rn1_W0[:128], r1(rn1_b0), rn1_W1[:128], r1(rn1_b1)),
        n_outs=2)

    p1 = _stream_matmul(Di[0], x_in.reshape(4 * n, 32), block_rows=512)
    dv = p1.reshape(fc, 128)

    fq = _full_call(_mid_body, (fc,), (128,),
                    (dv, rn0_W1[128:], r1(rn0_b1)))

    p2 = _stream_matmul(DiA[0], fq.reshape(4 * fc, 32), block_rows=512)
    daf = p2.reshape(n, 128)

    tail = jnp.tile(inp[:, -3:], (1, 40))
    out = _full_call(
        _post_body, (n,), (120,),
        (v2, daf, rn2_W0[:128], rn2_W0[128:], r1(rn2_b0),
         rn3_W0[:128], r1(rn3_b0), rn3_W1[:128], r1(rn3_b1),
         W2, r1(b2), tail))
    return out.reshape(1, n, 120)


# speedup vs baseline: 1.3701x; 1.1348x over previous
import jax
import jax.numpy as jnp
from jax.experimental import pallas as pl
from jax.experimental.pallas import tpu as pltpu


def _elu(x):
    return jnp.where(x > 0, x, jnp.exp(x) - 1.0)


def _bn(x):
    mu = jnp.mean(x, axis=0, keepdims=True)
    var = jnp.mean((x - mu) ** 2, axis=0, keepdims=True)
    return (x - mu) * jax.lax.rsqrt(var + 1e-5)


def _dot(a, b):
    return jax.lax.dot_general(a, b, (((1,), (0,)), ((), ())),
                               preferred_element_type=jnp.float32)


def _avg_block(v, w0, b0, w1, b1):
    h = _dot(_bn(_elu(v)), w0) + b0
    return v + _dot(_bn(_elu(h)), w1) + b1


def _to_q(x):
    m = x.shape[0]
    x3 = jnp.stack([x[:, 32 * j:32 * (j + 1)] for j in range(4)], axis=1)
    return x3.reshape(4 * m, 32)


def _from_q(p):
    m = p.shape[0] // 4
    p3 = p.reshape(m, 4, 32)
    return jnp.concatenate([p3[:, j, :] for j in range(4)], axis=1)


def _k1_body(di_ref, inp_ref, w1_ref, b1_ref, w0t_ref, b0_ref,
             a0_ref, a1_ref, a2_ref, a3_ref, dv_ref, v2_ref, xq_ref):
    @pl.when(pl.program_id(0) == 0)
    def _init():
        inp = inp_ref[...]
        w1 = w1_ref[...]
        v0 = (inp[:, 0:1] * w1[0:1, :] + inp[:, 1:2] * w1[1:2, :]
              + inp[:, 2:3] * w1[2:3, :] + b1_ref[...])
        x_in = _elu(v0)
        v1 = v0 + _dot(_bn(x_in), w0t_ref[...]) + b0_ref[...]
        v2_ref[...] = _avg_block(v1, a0_ref[...], a1_ref[...],
                                 a2_ref[...], a3_ref[...])
        xq_ref[...] = _to_q(x_in)

    dv_ref[...] = _from_q(_dot(di_ref[...], xq_ref[...]))


def _k2_body(da_ref, dv_ref, wm_ref, bm_ref, v2_ref, w2a_ref, w2b_ref,
             b2_ref, a0_ref, a1_ref, a2_ref, a3_ref, wo_ref, bo_ref,
             tail_ref, out_ref, fq_ref, daf_ref):
    i = pl.program_id(0)

    @pl.when(i == 0)
    def _init():
        fq = _elu(_dot(_bn(dv_ref[...]), wm_ref[...]) + bm_ref[...])
        fq_ref[...] = _to_q(fq)

    nb = da_ref.shape[0] // 4
    daf_ref[pl.ds(i * nb, nb), :] = _from_q(_dot(da_ref[...], fq_ref[...]))

    @pl.when(i == pl.num_programs(0) - 1)
    def _fin():
        v2 = v2_ref[...]
        x = (_dot(_bn(_elu(v2)), w2a_ref[...])
             + _dot(_bn(daf_ref[...]), w2b_ref[...]))
        v3 = v2 + x + b2_ref[...]
        v4 = _avg_block(v3, a0_ref[...], a1_ref[...], a2_ref[...], a3_ref[...])
        out_ref[...] = (_dot(_bn(_elu(v4)), wo_ref[...]) + bo_ref[...]
                        + tail_ref[...])


def _const_spec(shape):
    return pl.BlockSpec(shape, lambda i: tuple(0 for _ in shape))


def kernel(Di, DiA, mask, inputs, W1, b1, rn0_W0, rn0_b0, rn0_W1, rn0_b1,
           rn1_W0, rn1_b0, rn1_W1, rn1_b1, rn2_W0, rn2_b0, rn2_W1, rn2_b1,
           rn3_W0, rn3_b0, rn3_W1, rn3_b1, W2, b2):
    del mask
    n = inputs.shape[1]
    fc = Di.shape[1] // 4
    k1 = Di.shape[2]
    k2 = DiA.shape[2]
    br = 512
    br2 = 256
    inp = inputs[0]
    r1 = lambda v: v.reshape(1, -1)

    dv, v2 = pl.pallas_call(
        _k1_body,
        grid=(4 * fc // br,),
        in_specs=[
            pl.BlockSpec((br, k1), lambda i: (i, 0)),
            _const_spec((n, 3)),
            _const_spec((3, 128)),
            _const_spec((1, 128)),
            _const_spec((128, 128)),
            _const_spec((1, 128)),
            _const_spec((128, 128)),
            _const_spec((1, 128)),
            _const_spec((128, 128)),
            _const_spec((1, 128)),
        ],
        out_specs=[
            pl.BlockSpec((br // 4, 128), lambda i: (i, 0)),
            _const_spec((n, 128)),
        ],
        out_shape=[
            jax.ShapeDtypeStruct((fc, 128), jnp.float32),
            jax.ShapeDtypeStruct((n, 128), jnp.float32),
        ],
        scratch_shapes=[pltpu.VMEM((k1, 32), jnp.float32)],
    )(Di[0], inp, W1, r1(b1), rn0_W0[:128], r1(rn0_b0),
      rn1_W0[:128], r1(rn1_b0), rn1_W1[:128], r1(rn1_b1))

    tail = jnp.tile(inp[:, -3:], (1, 40))
    out = pl.pallas_call(
        _k2_body,
        grid=(4 * n // br2,),
        in_specs=[
            pl.BlockSpec((br2, k2), lambda i: (i, 0)),
            _const_spec((fc, 128)),
            _const_spec((128, 128)),
            _const_spec((1, 128)),
            _const_spec((n, 128)),
            _const_spec((128, 128)),
            _const_spec((128, 128)),
            _const_spec((1, 128)),
            _const_spec((128, 128)),
            _const_spec((1, 128)),
            _const_spec((128, 128)),
            _const_spec((1, 128)),
            _const_spec((128, 120)),
            _const_spec((1, 120)),
            _const_spec((n, 120)),
        ],
        out_specs=_const_spec((n, 120)),
        out_shape=jax.ShapeDtypeStruct((n, 120), jnp.float32),
        scratch_shapes=[pltpu.VMEM((k2, 32), jnp.float32),
                        pltpu.VMEM((n, 128), jnp.float32)],
    )(DiA[0], dv, rn0_W1[128:], r1(rn0_b1), v2,
      rn2_W0[:128], rn2_W0[128:], r1(rn2_b0),
      rn3_W0[:128], r1(rn3_b0), rn3_W1[:128], r1(rn3_b1),
      W2, r1(b2), tail)
    return out.reshape(1, n, 120)
